# baseline (speedup 1.0000x reference)
import jax
import jax.numpy as jnp
from jax.experimental import pallas as pl
from jax.experimental.pallas import tpu as pltpu

_HEADS = 4


def _chunk(ne):
    for c in range(4096, 0, -1):
        if ne % c == 0:
            return c
    return ne


def _mm_body(a_ref, b_ref, o_ref):
    o_ref[...] = jnp.dot(a_ref[...], b_ref[...],
                         preferred_element_type=jnp.float32)


def _mm(a, b, bm=1000, bn=512):
    m, k = a.shape
    _, n = b.shape
    bm = min(bm, m)
    bn = min(bn, n)
    return pl.pallas_call(
        _mm_body,
        grid=(m // bm, n // bn),
        in_specs=[pl.BlockSpec((bm, k), lambda i, j: (i, 0)),
                  pl.BlockSpec((k, bn), lambda i, j: (0, j))],
        out_specs=pl.BlockSpec((bm, bn), lambda i, j: (i, j)),
        out_shape=jax.ShapeDtypeStruct((m, n), jnp.float32),
    )(a, b)


def _deg_body(col_ref, deg_ref):
    i = pl.program_id(0)

    @pl.when(i == 0)
    def _init():
        deg_ref[...] = jnp.zeros_like(deg_ref)

    ne = col_ref.shape[0]

    def count(e, _):
        c = col_ref[e, 0]
        cur = deg_ref[pl.ds(c, 1), :]
        deg_ref[pl.ds(c, 1), :] = cur + 1.0
        return 0

    jax.lax.fori_loop(0, ne, count, 0)

    @pl.when(i == pl.num_programs(0) - 1)
    def _fin():
        d = deg_ref[...]
        deg_ref[...] = jnp.where(d > 0, jax.lax.rsqrt(d), 0.0)


def _deg(col, n):
    ne = col.shape[0]
    ec = _chunk(ne)
    return pl.pallas_call(
        _deg_body,
        grid=(ne // ec,),
        in_specs=[pl.BlockSpec((ec, 1), lambda i: (i, 0))],
        out_specs=pl.BlockSpec((n, 1), lambda i: (0, 0)),
        out_shape=jax.ShapeDtypeStruct((n, 1), jnp.float32),
    )(col)


def _norm_body(row_ref, col_ref, dinv_ref, norm_ref):
    ne = row_ref.shape[0]

    def body(e, _):
        r = row_ref[e, 0]
        c = col_ref[e, 0]
        dr = dinv_ref[pl.ds(r, 1), :]
        dc = dinv_ref[pl.ds(c, 1), :]
        norm_ref[pl.ds(e, 1), :] = dr * dc
        return 0

    jax.lax.fori_loop(0, ne, body, 0)


def _norm(row, col, dinv):
    ne = row.shape[0]
    n = dinv.shape[0]
    ec = _chunk(ne)
    return pl.pallas_call(
        _norm_body,
        grid=(ne // ec,),
        in_specs=[pl.BlockSpec((ec, 1), lambda i: (i, 0)),
                  pl.BlockSpec((ec, 1), lambda i: (i, 0)),
                  pl.BlockSpec((n, 1), lambda i: (0, 0))],
        out_specs=pl.BlockSpec((ec, 1), lambda i: (i, 0)),
        out_shape=jax.ShapeDtypeStruct((ne, 1), jnp.float32),
    )(row, col, dinv)


def _gcn_scatter_body(row_ref, col_ref, norm_ref, xw_ref, b_ref, o_ref,
                      *, relu_bias):
    i = pl.program_id(0)

    @pl.when(i == 0)
    def _init():
        o_ref[...] = jnp.zeros_like(o_ref)

    ne = row_ref.shape[0]

    def body(e, _):
        r = row_ref[e, 0]
        c = col_ref[e, 0]
        w = norm_ref[pl.ds(e, 1), :]
        src = xw_ref[pl.ds(r, 1), :]
        cur = o_ref[pl.ds(c, 1), :]
        o_ref[pl.ds(c, 1), :] = cur + src * w
        return 0

    jax.lax.fori_loop(0, ne, body, 0)

    if relu_bias:
        @pl.when(i == pl.num_programs(0) - 1)
        def _fin():
            o_ref[...] = jnp.maximum(o_ref[...] + b_ref[...], 0.0)


def _gcn_scatter(row, col, norm, xw, b, relu_bias=True):
    n, f = xw.shape
    ne = row.shape[0]
    ec = _chunk(ne)
    import functools
    return pl.pallas_call(
        functools.partial(_gcn_scatter_body, relu_bias=relu_bias),
        grid=(ne // ec,),
        in_specs=[pl.BlockSpec((ec, 1), lambda i: (i, 0)),
                  pl.BlockSpec((ec, 1), lambda i: (i, 0)),
                  pl.BlockSpec((ec, 1), lambda i: (i, 0)),
                  pl.BlockSpec((n, f), lambda i: (0, 0)),
                  pl.BlockSpec((1, f), lambda i: (0, 0))],
        out_specs=pl.BlockSpec((n, f), lambda i: (0, 0)),
        out_shape=jax.ShapeDtypeStruct((n, f), jnp.float32),
    )(row, col, norm, xw, b.reshape(1, f))


def _score_body(xg_ref, asrc_ref, adst_ref, s_ref, d_ref):
    ch = asrc_ref.shape[1]
    for h in range(_HEADS):
        blk = xg_ref[:, h * ch:(h + 1) * ch]
        s_ref[:, h] = jnp.sum(blk * asrc_ref[h:h + 1, :], axis=1)
        d_ref[:, h] = jnp.sum(blk * adst_ref[h:h + 1, :], axis=1)


def _att_scores(xg, att_src, att_dst):
    n = xg.shape[0]
    bm = 1000
    out_shape = jax.ShapeDtypeStruct((n, _HEADS), jnp.float32)
    return pl.pallas_call(
        _score_body,
        grid=(n // bm,),
        in_specs=[pl.BlockSpec((bm, xg.shape[1]), lambda i: (i, 0)),
                  pl.BlockSpec(att_src.shape, lambda i: (0, 0)),
                  pl.BlockSpec(att_dst.shape, lambda i: (0, 0))],
        out_specs=[pl.BlockSpec((bm, _HEADS), lambda i: (i, 0)),
                   pl.BlockSpec((bm, _HEADS), lambda i: (i, 0))],
        out_shape=[out_shape, out_shape],
    )(xg, att_src, att_dst)


def _att1_body(row_ref, col_ref, asrc_ref, adst_ref, ee_ref, emax_ref):
    i = pl.program_id(0)

    @pl.when(i == 0)
    def _init():
        emax_ref[...] = jnp.full_like(emax_ref, -1e30)

    ne = row_ref.shape[0]

    def body(e, _):
        r = row_ref[e, 0]
        c = col_ref[e, 0]
        v = asrc_ref[pl.ds(r, 1), :] + adst_ref[pl.ds(c, 1), :]
        v = jnp.where(v >= 0, v, 0.2 * v)
        ee_ref[pl.ds(e, 1), :] = v
        cur = emax_ref[pl.ds(c, 1), :]
        emax_ref[pl.ds(c, 1), :] = jnp.maximum(cur, v)
        return 0

    jax.lax.fori_loop(0, ne, body, 0)


def _att2_body(col_ref, ee_ref, emax_ref, ex_ref, den_ref):
    i = pl.program_id(0)

    @pl.when(i == 0)
    def _init():
        den_ref[...] = jnp.zeros_like(den_ref)

    ne = col_ref.shape[0]

    def body(e, _):
        c = col_ref[e, 0]
        m = emax_ref[pl.ds(c, 1), :]
        v = jnp.exp(ee_ref[pl.ds(e, 1), :] - m)
        ex_ref[pl.ds(e, 1), :] = v
        cur = den_ref[pl.ds(c, 1), :]
        den_ref[pl.ds(c, 1), :] = cur + v
        return 0

    jax.lax.fori_loop(0, ne, body, 0)


def _att3_body(col_ref, ex_ref, den_ref, alpha_ref):
    ne = col_ref.shape[0]

    def body(e, _):
        c = col_ref[e, 0]
        dn = den_ref[pl.ds(c, 1), :]
        v = ex_ref[pl.ds(e, 1), :]
        alpha_ref[pl.ds(e, 1), :] = v / (_HEADS * dn + _HEADS * 1e-16)
        return 0

    jax.lax.fori_loop(0, ne, body, 0)


def _att(row, col, asrc, adst, n):
    ne = row.shape[0]
    ec = _chunk(ne)
    nc = ne // ec
    espec = pl.BlockSpec((ec, 1), lambda i: (i, 0))
    hspec = pl.BlockSpec((ec, _HEADS), lambda i: (i, 0))
    nspec = pl.BlockSpec((n, _HEADS), lambda i: (0, 0))
    eshape = jax.ShapeDtypeStruct((ne, _HEADS), jnp.float32)
    nshape = jax.ShapeDtypeStruct((n, _HEADS), jnp.float32)

    ee, emax = pl.pallas_call(
        _att1_body, grid=(nc,),
        in_specs=[espec, espec, nspec, nspec],
        out_specs=[hspec, nspec],
        out_shape=[eshape, nshape],
    )(row, col, asrc, adst)

    ex, den = pl.pallas_call(
        _att2_body, grid=(nc,),
        in_specs=[espec, hspec, nspec],
        out_specs=[hspec, nspec],
        out_shape=[eshape, nshape],
    )(col, ee, emax)

    return pl.pallas_call(
        _att3_body, grid=(nc,),
        in_specs=[espec, hspec, nspec],
        out_specs=hspec,
        out_shape=eshape,
    )(col, ex, den)


def _add_body(a_ref, b_ref, o_ref):
    o_ref[...] = a_ref[...] + b_ref[...]


def _add2(a, b):
    n, f = a.shape
    bm = 1000
    return pl.pallas_call(
        _add_body,
        grid=(n // bm,),
        in_specs=[pl.BlockSpec((bm, f), lambda i: (i, 0)),
                  pl.BlockSpec((bm, f), lambda i: (i, 0))],
        out_specs=pl.BlockSpec((bm, f), lambda i: (i, 0)),
        out_shape=jax.ShapeDtypeStruct((n, f), jnp.float32),
    )(a, b)


def _head_body(h_ref, bg_ref, wc1_ref, bc1_ref, wc2_ref, bc2_ref,
               ws1_ref, bs1_ref, ws2_ref, bs2_ref, logit_ref, sev_ref):
    n = h_ref.shape[0]
    hr = jnp.maximum(h_ref[...] + bg_ref[...], 0.0)
    g = jnp.sum(hr, axis=0, keepdims=True) * (1.0 / n)
    hc = jnp.maximum(
        jnp.dot(g, wc1_ref[...], preferred_element_type=jnp.float32)
        + bc1_ref[...], 0.0)
    logit_ref[...] = jnp.dot(hc, wc2_ref[...],
                             preferred_element_type=jnp.float32) + bc2_ref[...]
    hs = jnp.maximum(
        jnp.dot(g, ws1_ref[...], preferred_element_type=jnp.float32)
        + bs1_ref[...], 0.0)
    sev_ref[...] = jax.nn.sigmoid(
        jnp.dot(hs, ws2_ref[...], preferred_element_type=jnp.float32)
        + bs2_ref[...])


def _heads(hgat, bg, Wc1, bc1, Wc2, bc2, Ws1, bs1, Ws2, bs2):
    f = hgat.shape[1]
    return pl.pallas_call(
        _head_body,
        out_shape=[jax.ShapeDtypeStruct((1, Wc2.shape[1]), jnp.float32),
                   jax.ShapeDtypeStruct((1, Ws2.shape[1]), jnp.float32)],
    )(hgat, bg.reshape(1, f), Wc1, bc1.reshape(1, -1), Wc2, bc2.reshape(1, -1),
      Ws1, bs1.reshape(1, -1), Ws2, bs2.reshape(1, -1))


def kernel(x, edge_index, W1, b1, W2, b2, Wg, att_src, att_dst, bg,
           Wc1, bc1, Wc2, bc2, Ws1, bs1, Ws2, bs2):
    n = x.shape[0]
    loop = jnp.arange(n, dtype=jnp.int32)
    row = jnp.concatenate([edge_index[0].astype(jnp.int32), loop]).reshape(-1, 1)
    col = jnp.concatenate([edge_index[1].astype(jnp.int32), loop]).reshape(-1, 1)

    dinv = _deg(col, n)
    norm = _norm(row, col, dinv)
    h = _gcn_scatter(row, col, norm, _mm(x, W1), b1)
    h = _gcn_scatter(row, col, norm, _mm(h, W2), b2)

    heads = att_src.shape[1]
    ch = att_src.shape[2]
    xg = _mm(h, Wg)
    asrc, adst = _att_scores(xg, att_src.reshape(heads, ch),
                             att_dst.reshape(heads, ch))
    alpha = _att(row, col, asrc, adst, n)
    outs = [_gcn_scatter(row, col, alpha[:, hh:hh + 1],
                         xg[:, hh * ch:(hh + 1) * ch],
                         bg, relu_bias=False)
            for hh in range(heads)]
    acc = _add2(_add2(outs[0], outs[1]), _add2(outs[2], outs[3]))

    logits, sev = _heads(acc, bg, Wc1, bc1, Wc2, bc2, Ws1, bs1, Ws2, bs2)
    return (logits, sev)
